# baseline (speedup 1.0000x reference)
import functools

import jax
import jax.numpy as jnp
from jax import lax
from jax.experimental import pallas as pl
from jax.experimental.pallas import tpu as pltpu
from jax.experimental.pallas import tpu_sc as plsc

_NS = 16
_NC = 2
_K = 128


def _round_up(x, m):
    return (x + m - 1) // m * m


def _make_deg_kernel(NPAD, EW):
    rows_per_sub = EW // _NS
    M = NPAD // _NS
    mesh = plsc.VectorSubcoreMesh(core_axis_name="c", subcore_axis_name="s")

    @functools.partial(
        pl.kernel,
        out_type=jax.ShapeDtypeStruct((2, NPAD), jnp.float32),
        mesh=mesh,
        scratch_types=[
            pltpu.VMEM_SHARED((_NS, NPAD), jnp.float32),
            pltpu.VMEM((rows_per_sub, _K), jnp.int32),
            pltpu.VMEM((NPAD,), jnp.float32),
            pltpu.VMEM((M,), jnp.float32),
            pltpu.VMEM((M,), jnp.float32),
        ],
        compiler_params=pltpu.CompilerParams(needs_layout_passes=False),
    )
    def deg_kernel(edges_hbm, out_hbm, stage_sh, idx_v, hist_v, red_v, tmp_v):
        c = lax.axis_index("c")
        s = lax.axis_index("s")

        def fill_zero(i, carry):
            hist_v[pl.ds(i * 16, 16)] = jnp.zeros((16,), jnp.float32)
            return carry

        lax.fori_loop(0, NPAD // 16, fill_zero, 0)
        pltpu.sync_copy(edges_hbm.at[c, pl.ds(s * rows_per_sub, rows_per_sub)],
                        idx_v)

        ones = jnp.full((16,), 1.0, jnp.float32)
        ng = _K // 16

        def body(i, carry):
            idx = idx_v[i // ng, pl.ds((i % ng) * 16, 16)]
            plsc.addupdate_scatter(hist_v, [idx], ones)
            return carry

        lax.fori_loop(0, rows_per_sub * ng, body, 0)
        pltpu.sync_copy(hist_v, stage_sh.at[s])
        plsc.subcore_barrier()

        def rz(i, carry):
            red_v[pl.ds(i * 16, 16)] = jnp.zeros((16,), jnp.float32)
            return carry

        lax.fori_loop(0, M // 16, rz, 0)
        for t in range(_NS):
            pltpu.sync_copy(stage_sh.at[t, pl.ds(s * M, M)], tmp_v)

            def racc(i, carry):
                sl = pl.ds(i * 16, 16)
                red_v[sl] = red_v[sl] + tmp_v[sl]
                return carry

            lax.fori_loop(0, M // 16, racc, 0)
        pltpu.sync_copy(red_v, out_hbm.at[c, pl.ds(s * M, M)])

    return deg_kernel


def _make_agg_kernel(NPAD, EW, D):
    NW = _NS * _NC
    cpw = EW // NW
    zrows = NPAD // _NS
    zc = zrows // _K
    assert zrows % _K == 0
    mesh = plsc.VectorSubcoreMesh(core_axis_name="c", subcore_axis_name="s")

    @functools.partial(
        pl.kernel,
        out_type=jax.ShapeDtypeStruct((_NC, NPAD, D), jnp.float32),
        mesh=mesh,
        scratch_types=[
            pltpu.VMEM_SHARED((NPAD, D), jnp.float32),
            pltpu.VMEM((cpw // 2, _K), jnp.int32),
            pltpu.VMEM((cpw // 2, _K), jnp.int32),
            pltpu.VMEM((_K, D), jnp.float32),
            pltpu.VMEM((_K, D), jnp.float32),
            pltpu.SemaphoreType.DMA,
            pltpu.SemaphoreType.DMA,
        ],
    )
    def agg_kernel(h_hbm, edges_hbm, out_hbm, agg_sh, sidx_v, didx_v,
                   ga_v, gb_v, sema, semb):
        c = lax.axis_index("c")
        s = lax.axis_index("s")
        w = s * _NC + c

        nk = D // 16

        def fill_zero(i, carry):
            ga_v[i // nk, pl.ds((i % nk) * 16, 16)] = jnp.zeros((16,), jnp.float32)
            return carry

        lax.fori_loop(0, _K * nk, fill_zero, 0)
        for t in range(zc):
            pltpu.sync_copy(ga_v, agg_sh.at[pl.ds(s * zrows + t * _K, _K)])
        plsc.subcore_barrier()

        hc = cpw // 2
        nt = hc // 2
        for p in range(2):
            pltpu.sync_copy(edges_hbm.at[0, pl.ds(w * cpw + p * hc, hc)], sidx_v)
            pltpu.sync_copy(edges_hbm.at[1, pl.ds(w * cpw + p * hc, hc)], didx_v)
            pltpu.async_copy(h_hbm.at[sidx_v.at[0]], ga_v, sema, priority=1)

            def body(t, carry):
                j0 = 2 * t
                pltpu.async_copy(h_hbm.at[sidx_v.at[j0 + 1]], gb_v, semb, priority=1)
                pltpu.make_async_copy(h_hbm.at[sidx_v.at[j0]], ga_v, sema).wait()
                pltpu.sync_copy(ga_v, agg_sh.at[didx_v.at[j0]], add=True)

                @pl.when(t + 1 < nt)
                def _():
                    pltpu.async_copy(h_hbm.at[sidx_v.at[j0 + 2]], ga_v, sema, priority=1)

                pltpu.make_async_copy(h_hbm.at[sidx_v.at[j0 + 1]], gb_v, semb).wait()
                pltpu.sync_copy(gb_v, agg_sh.at[didx_v.at[j0 + 1]], add=True)
                return carry

            lax.fori_loop(0, nt, body, 0)
        plsc.subcore_barrier()
        pltpu.sync_copy(agg_sh.at[pl.ds(s * zrows, zrows)],
                        out_hbm.at[c, pl.ds(s * zrows, zrows)])

    return agg_kernel


_BM = 512


def _tc1(features_p, W1, degtab):
    NPAD, DIN = features_p.shape
    DH = W1.shape[1]

    def body(f_ref, w_ref, d_ref, o_ref):
        h = jnp.dot(f_ref[...], w_ref[...], preferred_element_type=jnp.float32)
        norm = lax.rsqrt(jnp.maximum(d_ref[0], 1.0))[:, None]
        o_ref[...] = h * norm

    return pl.pallas_call(
        body,
        grid=(NPAD // _BM,),
        in_specs=[
            pl.BlockSpec((_BM, DIN), lambda m: (m, 0)),
            pl.BlockSpec((DIN, DH), lambda m: (0, 0)),
            pl.BlockSpec((2, _BM), lambda m: (0, m)),
        ],
        out_specs=pl.BlockSpec((_BM, DH), lambda m: (m, 0)),
        out_shape=jax.ShapeDtypeStruct((NPAD, DH), jnp.float32),
    )(features_p, W1, degtab)


def _tc2(partials, degtab, b1, W2):
    _, NPAD, DH = partials.shape
    DO = W2.shape[1]

    def body(p_ref, d_ref, b_ref, w_ref, o_ref):
        x = p_ref[0] + p_ref[1]
        norm_in = lax.rsqrt(jnp.maximum(d_ref[1], 1.0))[:, None]
        norm_out = lax.rsqrt(jnp.maximum(d_ref[0], 1.0))[:, None]
        h = jnp.maximum(x * norm_in + b_ref[...], 0.0)
        h = jnp.dot(h, w_ref[...], preferred_element_type=jnp.float32)
        o_ref[...] = jnp.pad(h * norm_out, ((0, 0), (0, DH - DO)))

    return pl.pallas_call(
        body,
        grid=(NPAD // _BM,),
        in_specs=[
            pl.BlockSpec((2, _BM, DH), lambda m: (0, m, 0)),
            pl.BlockSpec((2, _BM), lambda m: (0, m)),
            pl.BlockSpec((1, DH), lambda m: (0, 0)),
            pl.BlockSpec((DH, DO), lambda m: (0, 0)),
        ],
        out_specs=pl.BlockSpec((_BM, DH), lambda m: (m, 0)),
        out_shape=jax.ShapeDtypeStruct((NPAD, DH), jnp.float32),
    )(partials, degtab, b1, W2)


def _tc3(partials, degtab, b2, DO):
    _, NPAD, DH = partials.shape

    def body(p_ref, d_ref, b_ref, o_ref):
        x = (p_ref[0] + p_ref[1])[:, :DO]
        norm_in = lax.rsqrt(jnp.maximum(d_ref[1], 1.0))[:, None]
        o_ref[...] = x * norm_in + b_ref[...]

    return pl.pallas_call(
        body,
        grid=(NPAD // _BM,),
        in_specs=[
            pl.BlockSpec((2, _BM, DH), lambda m: (0, m, 0)),
            pl.BlockSpec((2, _BM), lambda m: (0, m)),
            pl.BlockSpec((1, DO), lambda m: (0, 0)),
        ],
        out_specs=pl.BlockSpec((_BM, DO), lambda m: (m, 0)),
        out_shape=jax.ShapeDtypeStruct((NPAD, DO), jnp.float32),
    )(partials, degtab, b2)


def kernel(features, edge_index, W1, b1, W2, b2):
    N, DIN = features.shape
    E = edge_index.shape[1]
    NPAD = _round_up(N + 1, 2560)
    EPAD = _round_up(E, _NS * _NC * _K * 8)
    EW = EPAD // _K

    edges = edge_index.astype(jnp.int32)
    pad = jnp.full((2, EPAD - E), N, dtype=jnp.int32)
    edges = jnp.concatenate([edges, pad], axis=1).reshape(2, EW, _K)

    features_p = jnp.zeros((NPAD, DIN), jnp.float32).at[:N].set(features)

    degtab = _make_deg_kernel(NPAD, EW)(edges)
    h1 = _tc1(features_p, W1, degtab)
    p1 = _make_agg_kernel(NPAD, EW, W1.shape[1])(h1, edges)
    h2 = _tc2(p1, degtab, b1.reshape(1, -1), W2)
    p2 = _make_agg_kernel(NPAD, EW, h2.shape[1])(h2, edges)
    out = _tc3(p2, degtab, b2.reshape(1, -1), W2.shape[1])
    return out[:N]


# speedup vs baseline: 1.0004x; 1.0004x over previous
import functools

import jax
import jax.numpy as jnp
from jax import lax
from jax.experimental import pallas as pl
from jax.experimental.pallas import tpu as pltpu
from jax.experimental.pallas import tpu_sc as plsc

_NS = 16
_NC = 2
_K = 128


def _round_up(x, m):
    return (x + m - 1) // m * m


def _make_deg_kernel(NPAD, EW):
    rows_per_sub = EW // _NS
    M = NPAD // _NS
    mesh = plsc.VectorSubcoreMesh(core_axis_name="c", subcore_axis_name="s")

    @functools.partial(
        pl.kernel,
        out_type=jax.ShapeDtypeStruct((2, NPAD), jnp.float32),
        mesh=mesh,
        scratch_types=[
            pltpu.VMEM_SHARED((_NS, NPAD), jnp.float32),
            pltpu.VMEM((rows_per_sub, _K), jnp.int32),
            pltpu.VMEM((NPAD,), jnp.float32),
            pltpu.VMEM((M,), jnp.float32),
            pltpu.VMEM((M,), jnp.float32),
        ],
        compiler_params=pltpu.CompilerParams(needs_layout_passes=False),
    )
    def deg_kernel(edges_hbm, out_hbm, stage_sh, idx_v, hist_v, red_v, tmp_v):
        c = lax.axis_index("c")
        s = lax.axis_index("s")

        def fill_zero(i, carry):
            hist_v[pl.ds(i * 16, 16)] = jnp.zeros((16,), jnp.float32)
            return carry

        lax.fori_loop(0, NPAD // 16, fill_zero, 0)
        pltpu.sync_copy(edges_hbm.at[c, pl.ds(s * rows_per_sub, rows_per_sub)],
                        idx_v)

        ones = jnp.full((16,), 1.0, jnp.float32)
        ng = _K // 16

        def body(i, carry):
            idx = idx_v[i // ng, pl.ds((i % ng) * 16, 16)]
            plsc.addupdate_scatter(hist_v, [idx], ones)
            return carry

        lax.fori_loop(0, rows_per_sub * ng, body, 0)
        pltpu.sync_copy(hist_v, stage_sh.at[s])
        plsc.subcore_barrier()

        def rz(i, carry):
            red_v[pl.ds(i * 16, 16)] = jnp.zeros((16,), jnp.float32)
            return carry

        lax.fori_loop(0, M // 16, rz, 0)
        for t in range(_NS):
            pltpu.sync_copy(stage_sh.at[t, pl.ds(s * M, M)], tmp_v)

            def racc(i, carry):
                sl = pl.ds(i * 16, 16)
                red_v[sl] = red_v[sl] + tmp_v[sl]
                return carry

            lax.fori_loop(0, M // 16, racc, 0)
        pltpu.sync_copy(red_v, out_hbm.at[c, pl.ds(s * M, M)])

    return deg_kernel


def _make_agg_kernel(NPAD, EW, D):
    NW = _NS * _NC
    cpw = EW // NW
    zrows = NPAD // _NS
    zc = zrows // _K
    assert zrows % _K == 0
    mesh = plsc.VectorSubcoreMesh(core_axis_name="c", subcore_axis_name="s")

    @functools.partial(
        pl.kernel,
        out_type=jax.ShapeDtypeStruct((_NC, NPAD, D), jnp.float32),
        mesh=mesh,
        scratch_types=[
            pltpu.VMEM_SHARED((NPAD, D), jnp.float32),
            pltpu.VMEM((cpw // 2, _K), jnp.int32),
            pltpu.VMEM((cpw // 2, _K), jnp.int32),
            pltpu.VMEM((_K, D), jnp.float32),
            pltpu.VMEM((_K, D), jnp.float32),
            pltpu.SemaphoreType.DMA,
            pltpu.SemaphoreType.DMA,
        ],
    )
    def agg_kernel(h_hbm, edges_hbm, out_hbm, agg_sh, sidx_v, didx_v,
                   ga_v, gb_v, sema, semb):
        c = lax.axis_index("c")
        s = lax.axis_index("s")
        w = s * _NC + c

        nk = D // 16

        def fill_zero(i, carry):
            ga_v[i // nk, pl.ds((i % nk) * 16, 16)] = jnp.zeros((16,), jnp.float32)
            return carry

        lax.fori_loop(0, _K * nk, fill_zero, 0)
        for t in range(zc):
            pltpu.sync_copy(ga_v, agg_sh.at[pl.ds(s * zrows + t * _K, _K)])
        plsc.subcore_barrier()

        hc = cpw // 2
        nt = hc // 2
        for p in range(2):
            pltpu.sync_copy(edges_hbm.at[0, pl.ds(w * cpw + p * hc, hc)], sidx_v)
            pltpu.sync_copy(edges_hbm.at[1, pl.ds(w * cpw + p * hc, hc)], didx_v)
            pltpu.async_copy(h_hbm.at[sidx_v.at[0]], ga_v, sema)

            def body(t, carry):
                j0 = 2 * t
                pltpu.async_copy(h_hbm.at[sidx_v.at[j0 + 1]], gb_v, semb)
                pltpu.make_async_copy(h_hbm.at[sidx_v.at[j0]], ga_v, sema).wait()
                pltpu.sync_copy(ga_v, agg_sh.at[didx_v.at[j0]], add=True)

                @pl.when(t + 1 < nt)
                def _():
                    pltpu.async_copy(h_hbm.at[sidx_v.at[j0 + 2]], ga_v, sema)

                pltpu.make_async_copy(h_hbm.at[sidx_v.at[j0 + 1]], gb_v, semb).wait()
                pltpu.sync_copy(gb_v, agg_sh.at[didx_v.at[j0 + 1]], add=True)
                return carry

            lax.fori_loop(0, nt, body, 0)
        plsc.subcore_barrier()
        pltpu.sync_copy(agg_sh.at[pl.ds(s * zrows, zrows)],
                        out_hbm.at[c, pl.ds(s * zrows, zrows)])

    return agg_kernel


_BM = 512


def _tc1(features_p, W1, degtab):
    NPAD, DIN = features_p.shape
    DH = W1.shape[1]

    def body(f_ref, w_ref, d_ref, o_ref):
        h = jnp.dot(f_ref[...], w_ref[...], preferred_element_type=jnp.float32)
        norm = lax.rsqrt(jnp.maximum(d_ref[0], 1.0))[:, None]
        o_ref[...] = h * norm

    return pl.pallas_call(
        body,
        grid=(NPAD // _BM,),
        in_specs=[
            pl.BlockSpec((_BM, DIN), lambda m: (m, 0)),
            pl.BlockSpec((DIN, DH), lambda m: (0, 0)),
            pl.BlockSpec((2, _BM), lambda m: (0, m)),
        ],
        out_specs=pl.BlockSpec((_BM, DH), lambda m: (m, 0)),
        out_shape=jax.ShapeDtypeStruct((NPAD, DH), jnp.float32),
    )(features_p, W1, degtab)


def _tc2(partials, degtab, b1, W2):
    _, NPAD, DH = partials.shape
    DO = W2.shape[1]

    def body(p_ref, d_ref, b_ref, w_ref, o_ref):
        x = p_ref[0] + p_ref[1]
        norm_in = lax.rsqrt(jnp.maximum(d_ref[1], 1.0))[:, None]
        norm_out = lax.rsqrt(jnp.maximum(d_ref[0], 1.0))[:, None]
        h = jnp.maximum(x * norm_in + b_ref[...], 0.0)
        h = jnp.dot(h, w_ref[...], preferred_element_type=jnp.float32)
        o_ref[...] = jnp.pad(h * norm_out, ((0, 0), (0, DH - DO)))

    return pl.pallas_call(
        body,
        grid=(NPAD // _BM,),
        in_specs=[
            pl.BlockSpec((2, _BM, DH), lambda m: (0, m, 0)),
            pl.BlockSpec((2, _BM), lambda m: (0, m)),
            pl.BlockSpec((1, DH), lambda m: (0, 0)),
            pl.BlockSpec((DH, DO), lambda m: (0, 0)),
        ],
        out_specs=pl.BlockSpec((_BM, DH), lambda m: (m, 0)),
        out_shape=jax.ShapeDtypeStruct((NPAD, DH), jnp.float32),
    )(partials, degtab, b1, W2)


def _tc3(partials, degtab, b2, DO):
    _, NPAD, DH = partials.shape

    def body(p_ref, d_ref, b_ref, o_ref):
        x = (p_ref[0] + p_ref[1])[:, :DO]
        norm_in = lax.rsqrt(jnp.maximum(d_ref[1], 1.0))[:, None]
        o_ref[...] = x * norm_in + b_ref[...]

    return pl.pallas_call(
        body,
        grid=(NPAD // _BM,),
        in_specs=[
            pl.BlockSpec((2, _BM, DH), lambda m: (0, m, 0)),
            pl.BlockSpec((2, _BM), lambda m: (0, m)),
            pl.BlockSpec((1, DO), lambda m: (0, 0)),
        ],
        out_specs=pl.BlockSpec((_BM, DO), lambda m: (m, 0)),
        out_shape=jax.ShapeDtypeStruct((NPAD, DO), jnp.float32),
    )(partials, degtab, b2)


def kernel(features, edge_index, W1, b1, W2, b2):
    N, DIN = features.shape
    E = edge_index.shape[1]
    NPAD = _round_up(N + 1, 2560)
    EPAD = _round_up(E, _NS * _NC * _K * 8)
    EW = EPAD // _K

    edges = edge_index.astype(jnp.int32)
    pad = jnp.full((2, EPAD - E), N, dtype=jnp.int32)
    edges = jnp.concatenate([edges, pad], axis=1).reshape(2, EW, _K)

    features_p = jnp.zeros((NPAD, DIN), jnp.float32).at[:N].set(features)

    degtab = _make_deg_kernel(NPAD, EW)(edges)
    h1 = _tc1(features_p, W1, degtab)
    p1 = _make_agg_kernel(NPAD, EW, W1.shape[1])(h1, edges)
    h2 = _tc2(p1, degtab, b1.reshape(1, -1), W2)
    p2 = _make_agg_kernel(NPAD, EW, h2.shape[1])(h2, edges)
    out = _tc3(p2, degtab, b2.reshape(1, -1), W2.shape[1])
    return out[:N]
